# Initial kernel scaffold; baseline (speedup 1.0000x reference)
import jax
import jax.numpy as jnp
from jax.experimental import pallas as pl


def kernel(x, from_logits, E):
    raise NotImplementedError("write your pallas kernel here")


# speedup vs baseline: 3.6698x; 3.6698x over previous
import functools

import jax
import jax.numpy as jnp
from jax import lax
from jax.experimental import pallas as pl
from jax.experimental.pallas import tpu as pltpu
from jax.experimental.pallas import tpu_sc as plsc

VOCAB = 100000
D_MODEL = 512
BATCH = 64

V_BLK = 8192
N_BLK = (VOCAB + V_BLK - 1) // V_BLK

_INT_MAX = 2**31 - 1


def _argmax_body(x_ref, idx_out, s_out, m_ref, mi_ref):
    j = pl.program_id(0)
    xb = x_ref[...]
    col = j * V_BLK + lax.broadcasted_iota(jnp.int32, (BATCH, V_BLK), 1)
    xb = jnp.where(col < VOCAB, xb, -jnp.inf)
    bmax = jnp.max(xb, axis=1, keepdims=True)
    bidx = jnp.min(jnp.where(xb == bmax, col, _INT_MAX), axis=1, keepdims=True)

    @pl.when(j == 0)
    def _():
        m_ref[...] = bmax
        mi_ref[...] = bidx

    @pl.when(j > 0)
    def _():
        better = bmax > m_ref[...]
        m_ref[...] = jnp.where(better, bmax, m_ref[...])
        mi_ref[...] = jnp.where(better, bidx, mi_ref[...])

    @pl.when(j == N_BLK - 1)
    def _():
        m = m_ref[...]
        idx_out[...] = jnp.broadcast_to(mi_ref[...], (BATCH, 128))
        s_out[...] = jnp.broadcast_to((1.0 - m) + m, (BATCH, 128))


_argmax_call = pl.pallas_call(
    _argmax_body,
    grid=(N_BLK,),
    in_specs=[pl.BlockSpec((BATCH, V_BLK), lambda j: (0, j))],
    out_specs=[
        pl.BlockSpec((BATCH, 128), lambda j: (0, 0)),
        pl.BlockSpec((BATCH, 128), lambda j: (0, 0)),
    ],
    out_shape=[
        jax.ShapeDtypeStruct((BATCH, 128), jnp.int32),
        jax.ShapeDtypeStruct((BATCH, 128), jnp.float32),
    ],
    scratch_shapes=[
        pltpu.VMEM((BATCH, 1), jnp.float32),
        pltpu.VMEM((BATCH, 1), jnp.int32),
    ],
)

_N_WORKERS = 8
_RPW = BATCH // _N_WORKERS


@functools.lru_cache(maxsize=None)
def _make_gather_scale():
    @functools.partial(
        pl.kernel,
        mesh=plsc.VectorSubcoreMesh(core_axis_name="c", subcore_axis_name="s"),
        out_type=jax.ShapeDtypeStruct((BATCH, D_MODEL), jnp.float32),
        scratch_types=[
            pltpu.VMEM((_RPW,), jnp.int32),
            pltpu.VMEM((_RPW, 16), jnp.float32),
            pltpu.VMEM((_RPW, D_MODEL), jnp.float32),
            pltpu.SemaphoreType.DMA,
        ],
    )
    def _gather_scale(e_hbm, idx_hbm, s_hbm, out_hbm, idx_v, s_v, rows_v, sem):
        info = plsc.get_sparse_core_info()
        wid = lax.axis_index("s") * info.num_cores + lax.axis_index("c")

        @pl.when(wid < _N_WORKERS)
        def _():
            base = wid * _RPW
            pltpu.sync_copy(idx_hbm.at[pl.ds(base, _RPW)], idx_v)
            pltpu.sync_copy(s_hbm.at[pl.ds(base, _RPW)], s_v)
            pltpu.async_copy(e_hbm.at[idx_v], rows_v, sem).wait()
            for i in range(_RPW):
                sbc = s_v[i]
                for jj in range(D_MODEL // 16):
                    sl = pl.ds(jj * 16, 16)
                    rows_v[i, sl] = rows_v[i, sl] * sbc
            pltpu.sync_copy(rows_v, out_hbm.at[pl.ds(base, _RPW)])

    return _gather_scale


def kernel(x, from_logits, E):
    idx128, s128 = _argmax_call(x)
    idx = idx128[:, 0]
    s16 = s128[:, :16]
    return _make_gather_scale()(E, idx, s16)


# speedup vs baseline: 3.8855x; 1.0588x over previous
import functools

import jax
import jax.numpy as jnp
from jax import lax
from jax.experimental import pallas as pl
from jax.experimental.pallas import tpu as pltpu
from jax.experimental.pallas import tpu_sc as plsc

VOCAB = 100000
D_MODEL = 512
BATCH = 64

V_BLK = 8192
N_BLK = (VOCAB + V_BLK - 1) // V_BLK
CHUNKS = V_BLK // 128
N_ACC = 4
LAST_FULL = (VOCAB - (N_BLK - 1) * V_BLK) // 128
LAST_REM = VOCAB - (N_BLK - 1) * V_BLK - LAST_FULL * 128

_BIG = float(1e9)
_NEG = float("-inf")


def _argmax_body(x_ref, idx_out, s_out, m_ref, c_ref):
    j = pl.program_id(0)

    def scan_chunks(n_full, mask_rem):
        m = [m_ref[a] for a in range(N_ACC)]
        c = [c_ref[a] for a in range(N_ACC)]
        n = n_full + (1 if mask_rem else 0)
        for k in range(n):
            a = k % N_ACC
            xc = x_ref[:, k * 128:(k + 1) * 128]
            if mask_rem and k == n_full:
                lane = lax.broadcasted_iota(jnp.int32, (BATCH, 128), 1)
                xc = jnp.where(lane < LAST_REM, xc, _NEG)
            cid = (j * CHUNKS + k).astype(jnp.float32)
            gt = xc > m[a]
            m[a] = jnp.where(gt, xc, m[a])
            c[a] = jnp.where(gt, cid, c[a])
        for a in range(N_ACC):
            m_ref[a] = m[a]
            c_ref[a] = c[a]

    @pl.when(j == 0)
    def _():
        for a in range(N_ACC):
            m_ref[a] = jnp.full((BATCH, 128), _NEG, jnp.float32)
            c_ref[a] = jnp.zeros((BATCH, 128), jnp.float32)

    @pl.when(j < N_BLK - 1)
    def _():
        scan_chunks(CHUNKS, False)

    @pl.when(j == N_BLK - 1)
    def _():
        scan_chunks(LAST_FULL, True)

        m, c = m_ref[0], c_ref[0]
        for a in range(1, N_ACC):
            ma, ca = m_ref[a], c_ref[a]
            take = (ma > m) | ((ma == m) & (ca < c))
            m = jnp.where(take, ma, m)
            c = jnp.where(take, ca, c)

        rmax = jnp.max(m, axis=1, keepdims=True)
        lane = lax.broadcasted_iota(jnp.int32, (BATCH, 128), 1).astype(
            jnp.float32)
        colf = c * 128.0 + lane
        colmin = jnp.min(jnp.where(m == rmax, colf, _BIG), axis=1,
                         keepdims=True)
        idx_out[...] = jnp.broadcast_to(colmin.astype(jnp.int32),
                                        (BATCH, 128))
        s_out[...] = jnp.broadcast_to((1.0 - rmax) + rmax, (BATCH, 128))


_argmax_call = pl.pallas_call(
    _argmax_body,
    grid=(N_BLK,),
    in_specs=[pl.BlockSpec((BATCH, V_BLK), lambda j: (0, j))],
    out_specs=[
        pl.BlockSpec((BATCH, 128), lambda j: (0, 0)),
        pl.BlockSpec((BATCH, 128), lambda j: (0, 0)),
    ],
    out_shape=[
        jax.ShapeDtypeStruct((BATCH, 128), jnp.int32),
        jax.ShapeDtypeStruct((BATCH, 128), jnp.float32),
    ],
    scratch_shapes=[
        pltpu.VMEM((N_ACC, BATCH, 128), jnp.float32),
        pltpu.VMEM((N_ACC, BATCH, 128), jnp.float32),
    ],
)

_N_WORKERS = 8
_RPW = BATCH // _N_WORKERS


@functools.lru_cache(maxsize=None)
def _make_gather_scale():
    @functools.partial(
        pl.kernel,
        mesh=plsc.VectorSubcoreMesh(core_axis_name="c", subcore_axis_name="s"),
        out_type=jax.ShapeDtypeStruct((BATCH, D_MODEL), jnp.float32),
        scratch_types=[
            pltpu.VMEM((_RPW,), jnp.int32),
            pltpu.VMEM((_RPW, 16), jnp.float32),
            pltpu.VMEM((_RPW, D_MODEL), jnp.float32),
            pltpu.SemaphoreType.DMA,
        ],
    )
    def _gather_scale(e_hbm, idx_hbm, s_hbm, out_hbm, idx_v, s_v, rows_v, sem):
        info = plsc.get_sparse_core_info()
        wid = lax.axis_index("s") * info.num_cores + lax.axis_index("c")

        @pl.when(wid < _N_WORKERS)
        def _():
            base = wid * _RPW
            pltpu.sync_copy(idx_hbm.at[pl.ds(base, _RPW)], idx_v)
            pltpu.sync_copy(s_hbm.at[pl.ds(base, _RPW)], s_v)
            pltpu.async_copy(e_hbm.at[idx_v], rows_v, sem).wait()
            for i in range(_RPW):
                sbc = s_v[i]
                for jj in range(D_MODEL // 16):
                    sl = pl.ds(jj * 16, 16)
                    rows_v[i, sl] = rows_v[i, sl] * sbc
            pltpu.sync_copy(rows_v, out_hbm.at[pl.ds(base, _RPW)])

    return _gather_scale


def kernel(x, from_logits, E):
    idx128, s128 = _argmax_call(x)
    idx = idx128[:, 0]
    s16 = s128[:, :16]
    return _make_gather_scale()(E, idx, s16)


# speedup vs baseline: 4.0193x; 1.0344x over previous
import functools

import jax
import jax.numpy as jnp
from jax import lax
from jax.experimental import pallas as pl
from jax.experimental.pallas import tpu as pltpu
from jax.experimental.pallas import tpu_sc as plsc

VOCAB = 100000
D_MODEL = 512
BATCH = 64

V_BLK = 8192
N_BLK = (VOCAB + V_BLK - 1) // V_BLK
CHUNKS = V_BLK // 128
N_ACC = 4
LAST_FULL = (VOCAB - (N_BLK - 1) * V_BLK) // 128
LAST_REM = VOCAB - (N_BLK - 1) * V_BLK - LAST_FULL * 128

_BIG = float(1e9)
_NEG = float("-inf")


def _argmax_body(x_ref, idx_out, s_out, m_ref, c_ref):
    j = pl.program_id(0)

    def scan_chunks(n_full, mask_rem):
        n = n_full + (1 if mask_rem else 0)
        for r in range(BATCH // 8):
            rs = pl.ds(r * 8, 8)
            m = [m_ref[a, rs, :] for a in range(N_ACC)]
            c = [c_ref[a, rs, :] for a in range(N_ACC)]
            for k in range(n):
                a = k % N_ACC
                xc = x_ref[rs, k * 128:(k + 1) * 128]
                if mask_rem and k == n_full:
                    lane = lax.broadcasted_iota(jnp.int32, (8, 128), 1)
                    xc = jnp.where(lane < LAST_REM, xc, _NEG)
                cid = (j * CHUNKS + k).astype(jnp.float32)
                gt = xc > m[a]
                m[a] = jnp.where(gt, xc, m[a])
                c[a] = jnp.where(gt, cid, c[a])
            for a in range(N_ACC):
                m_ref[a, rs, :] = m[a]
                c_ref[a, rs, :] = c[a]

    @pl.when(j == 0)
    def _():
        for a in range(N_ACC):
            m_ref[a] = jnp.full((BATCH, 128), _NEG, jnp.float32)
            c_ref[a] = jnp.zeros((BATCH, 128), jnp.float32)

    @pl.when(j < N_BLK - 1)
    def _():
        scan_chunks(CHUNKS, False)

    @pl.when(j == N_BLK - 1)
    def _():
        scan_chunks(LAST_FULL, True)

        m, c = m_ref[0], c_ref[0]
        for a in range(1, N_ACC):
            ma, ca = m_ref[a], c_ref[a]
            take = (ma > m) | ((ma == m) & (ca < c))
            m = jnp.where(take, ma, m)
            c = jnp.where(take, ca, c)

        rmax = jnp.max(m, axis=1, keepdims=True)
        lane = lax.broadcasted_iota(jnp.int32, (BATCH, 128), 1).astype(
            jnp.float32)
        colf = c * 128.0 + lane
        colmin = jnp.min(jnp.where(m == rmax, colf, _BIG), axis=1,
                         keepdims=True)
        idx_out[...] = jnp.broadcast_to(colmin.astype(jnp.int32),
                                        (BATCH, 128))
        s_out[...] = jnp.broadcast_to((1.0 - rmax) + rmax, (BATCH, 128))


_argmax_call = pl.pallas_call(
    _argmax_body,
    grid=(N_BLK,),
    in_specs=[pl.BlockSpec((BATCH, V_BLK), lambda j: (0, j))],
    out_specs=[
        pl.BlockSpec((BATCH, 128), lambda j: (0, 0)),
        pl.BlockSpec((BATCH, 128), lambda j: (0, 0)),
    ],
    out_shape=[
        jax.ShapeDtypeStruct((BATCH, 128), jnp.int32),
        jax.ShapeDtypeStruct((BATCH, 128), jnp.float32),
    ],
    scratch_shapes=[
        pltpu.VMEM((N_ACC, BATCH, 128), jnp.float32),
        pltpu.VMEM((N_ACC, BATCH, 128), jnp.float32),
    ],
)

_N_WORKERS = 8
_RPW = BATCH // _N_WORKERS


@functools.lru_cache(maxsize=None)
def _make_gather_scale():
    @functools.partial(
        pl.kernel,
        mesh=plsc.VectorSubcoreMesh(core_axis_name="c", subcore_axis_name="s"),
        out_type=jax.ShapeDtypeStruct((BATCH, D_MODEL), jnp.float32),
        scratch_types=[
            pltpu.VMEM((_RPW,), jnp.int32),
            pltpu.VMEM((_RPW, 16), jnp.float32),
            pltpu.VMEM((_RPW, D_MODEL), jnp.float32),
            pltpu.SemaphoreType.DMA,
        ],
    )
    def _gather_scale(e_hbm, idx_hbm, s_hbm, out_hbm, idx_v, s_v, rows_v, sem):
        info = plsc.get_sparse_core_info()
        wid = lax.axis_index("s") * info.num_cores + lax.axis_index("c")

        @pl.when(wid < _N_WORKERS)
        def _():
            base = wid * _RPW
            pltpu.sync_copy(idx_hbm.at[pl.ds(base, _RPW)], idx_v)
            pltpu.sync_copy(s_hbm.at[pl.ds(base, _RPW)], s_v)
            pltpu.async_copy(e_hbm.at[idx_v], rows_v, sem).wait()
            for i in range(_RPW):
                sbc = s_v[i]
                for jj in range(D_MODEL // 16):
                    sl = pl.ds(jj * 16, 16)
                    rows_v[i, sl] = rows_v[i, sl] * sbc
            pltpu.sync_copy(rows_v, out_hbm.at[pl.ds(base, _RPW)])

    return _gather_scale


def kernel(x, from_logits, E):
    idx128, s128 = _argmax_call(x)
    idx = idx128[:, 0]
    s16 = s128[:, :16]
    return _make_gather_scale()(E, idx, s16)


# speedup vs baseline: 4.1754x; 1.0388x over previous
import functools

import jax
import jax.numpy as jnp
from jax import lax
from jax.experimental import pallas as pl
from jax.experimental.pallas import tpu as pltpu
from jax.experimental.pallas import tpu_sc as plsc

VOCAB = 100000
D_MODEL = 512
BATCH = 64

V_BLK = 8192
N_BLK = (VOCAB + V_BLK - 1) // V_BLK
CHUNKS = V_BLK // 128
N_ACC = 4
LAST_FULL = (VOCAB - (N_BLK - 1) * V_BLK) // 128
LAST_REM = VOCAB - (N_BLK - 1) * V_BLK - LAST_FULL * 128

_BIG = float(1e9)
_NEG = float("-inf")


def _argmax_body(x_ref, idx_out, s_out, m_ref, c_ref):
    j = pl.program_id(0)

    def scan_chunks(n_full, mask_rem):
        n = n_full + (1 if mask_rem else 0)
        for r in range(BATCH // 8):
            rs = pl.ds(r * 8, 8)
            m = [m_ref[a, rs, :] for a in range(N_ACC)]
            c = [c_ref[a, rs, :] for a in range(N_ACC)]
            for k in range(n):
                a = k % N_ACC
                xc = x_ref[rs, k * 128:(k + 1) * 128]
                if mask_rem and k == n_full:
                    lane = lax.broadcasted_iota(jnp.int32, (8, 128), 1)
                    xc = jnp.where(lane < LAST_REM, xc, _NEG)
                cid = (j * CHUNKS + k).astype(jnp.float32)
                gt = xc > m[a]
                m[a] = jnp.where(gt, xc, m[a])
                c[a] = jnp.where(gt, cid, c[a])
            for a in range(N_ACC):
                m_ref[a, rs, :] = m[a]
                c_ref[a, rs, :] = c[a]

    @pl.when(j == 0)
    def _():
        for a in range(N_ACC):
            m_ref[a] = jnp.full((BATCH, 128), _NEG, jnp.float32)
            c_ref[a] = jnp.zeros((BATCH, 128), jnp.float32)

    @pl.when(j < N_BLK - 1)
    def _():
        scan_chunks(CHUNKS, False)

    @pl.when(j == N_BLK - 1)
    def _():
        scan_chunks(LAST_FULL, True)

        m, c = m_ref[0], c_ref[0]
        for a in range(1, N_ACC):
            ma, ca = m_ref[a], c_ref[a]
            take = (ma > m) | ((ma == m) & (ca < c))
            m = jnp.where(take, ma, m)
            c = jnp.where(take, ca, c)

        rmax = jnp.max(m, axis=1, keepdims=True)
        lane = lax.broadcasted_iota(jnp.int32, (BATCH, 128), 1).astype(
            jnp.float32)
        colf = c * 128.0 + lane
        colmin = jnp.min(jnp.where(m == rmax, colf, _BIG), axis=1,
                         keepdims=True)
        idx_out[...] = jnp.reshape(colmin.astype(jnp.int32), (BATCH,))
        s_out[...] = jnp.broadcast_to((1.0 - rmax) + rmax, (BATCH, 16))


_argmax_call = pl.pallas_call(
    _argmax_body,
    grid=(N_BLK,),
    in_specs=[pl.BlockSpec((BATCH, V_BLK), lambda j: (0, j))],
    out_specs=[
        pl.BlockSpec((BATCH,), lambda j: (0,)),
        pl.BlockSpec((BATCH, 16), lambda j: (0, 0)),
    ],
    out_shape=[
        jax.ShapeDtypeStruct((BATCH,), jnp.int32),
        jax.ShapeDtypeStruct((BATCH, 16), jnp.float32),
    ],
    scratch_shapes=[
        pltpu.VMEM((N_ACC, BATCH, 128), jnp.float32),
        pltpu.VMEM((N_ACC, BATCH, 128), jnp.float32),
    ],
)

_N_WORKERS = 8
_RPW = BATCH // _N_WORKERS


@functools.lru_cache(maxsize=None)
def _make_gather_scale():
    @functools.partial(
        pl.kernel,
        mesh=plsc.VectorSubcoreMesh(core_axis_name="c", subcore_axis_name="s"),
        out_type=jax.ShapeDtypeStruct((BATCH, D_MODEL), jnp.float32),
        scratch_types=[
            pltpu.VMEM((_RPW,), jnp.int32),
            pltpu.VMEM((_RPW, 16), jnp.float32),
            pltpu.VMEM((_RPW, D_MODEL), jnp.float32),
            pltpu.SemaphoreType.DMA,
        ],
    )
    def _gather_scale(e_hbm, idx_hbm, s_hbm, out_hbm, idx_v, s_v, rows_v, sem):
        info = plsc.get_sparse_core_info()
        wid = lax.axis_index("s") * info.num_cores + lax.axis_index("c")

        @pl.when(wid < _N_WORKERS)
        def _():
            base = wid * _RPW
            pltpu.sync_copy(idx_hbm.at[pl.ds(base, _RPW)], idx_v)
            pltpu.sync_copy(s_hbm.at[pl.ds(base, _RPW)], s_v)
            pltpu.async_copy(e_hbm.at[idx_v], rows_v, sem).wait()
            for i in range(_RPW):
                sbc = s_v[i]
                for jj in range(D_MODEL // 16):
                    sl = pl.ds(jj * 16, 16)
                    rows_v[i, sl] = rows_v[i, sl] * sbc
            pltpu.sync_copy(rows_v, out_hbm.at[pl.ds(base, _RPW)])

    return _gather_scale


def kernel(x, from_logits, E):
    idx, s16 = _argmax_call(x)
    return _make_gather_scale()(E, idx, s16)


# speedup vs baseline: 4.7184x; 1.1301x over previous
import functools

import jax
import jax.numpy as jnp
from jax import lax
from jax.experimental import pallas as pl
from jax.experimental.pallas import tpu as pltpu
from jax.experimental.pallas import tpu_sc as plsc

VOCAB = 100000
D_MODEL = 512
BATCH = 64

V_BLK = 16384
N_BLK = (VOCAB + V_BLK - 1) // V_BLK
CHUNKS = V_BLK // 128
N_ACC = 4
LAST_FULL = (VOCAB - (N_BLK - 1) * V_BLK) // 128
LAST_REM = VOCAB - (N_BLK - 1) * V_BLK - LAST_FULL * 128

_BIG = float(1e9)
_NEG = float("-inf")


def _argmax_body(x_ref, idx_out, s_out, m_ref, c_ref):
    j = pl.program_id(0)

    def scan_chunks(n_full, mask_rem):
        n = n_full + (1 if mask_rem else 0)
        for r in range(BATCH // 8):
            rs = pl.ds(r * 8, 8)
            m = [m_ref[a, rs, :] for a in range(N_ACC)]
            c = [c_ref[a, rs, :] for a in range(N_ACC)]
            for k in range(n):
                a = k % N_ACC
                xc = x_ref[rs, k * 128:(k + 1) * 128]
                if mask_rem and k == n_full:
                    lane = lax.broadcasted_iota(jnp.int32, (8, 128), 1)
                    xc = jnp.where(lane < LAST_REM, xc, _NEG)
                cid = (j * CHUNKS + k).astype(jnp.float32)
                gt = xc > m[a]
                m[a] = jnp.where(gt, xc, m[a])
                c[a] = jnp.where(gt, cid, c[a])
            for a in range(N_ACC):
                m_ref[a, rs, :] = m[a]
                c_ref[a, rs, :] = c[a]

    @pl.when(j == 0)
    def _():
        for a in range(N_ACC):
            m_ref[a] = jnp.full((BATCH, 128), _NEG, jnp.float32)
            c_ref[a] = jnp.zeros((BATCH, 128), jnp.float32)

    @pl.when(j < N_BLK - 1)
    def _():
        scan_chunks(CHUNKS, False)

    @pl.when(j == N_BLK - 1)
    def _():
        scan_chunks(LAST_FULL, True)

        m, c = m_ref[0], c_ref[0]
        for a in range(1, N_ACC):
            ma, ca = m_ref[a], c_ref[a]
            take = (ma > m) | ((ma == m) & (ca < c))
            m = jnp.where(take, ma, m)
            c = jnp.where(take, ca, c)

        rmax = jnp.max(m, axis=1, keepdims=True)
        lane = lax.broadcasted_iota(jnp.int32, (BATCH, 128), 1).astype(
            jnp.float32)
        colf = c * 128.0 + lane
        colmin = jnp.min(jnp.where(m == rmax, colf, _BIG), axis=1,
                         keepdims=True)
        idx_out[...] = jnp.reshape(colmin.astype(jnp.int32), (BATCH,))
        s_out[...] = jnp.broadcast_to((1.0 - rmax) + rmax, (BATCH, 16))


_argmax_call = pl.pallas_call(
    _argmax_body,
    grid=(N_BLK,),
    in_specs=[pl.BlockSpec((BATCH, V_BLK), lambda j: (0, j))],
    out_specs=[
        pl.BlockSpec((BATCH,), lambda j: (0,)),
        pl.BlockSpec((BATCH, 16), lambda j: (0, 0)),
    ],
    out_shape=[
        jax.ShapeDtypeStruct((BATCH,), jnp.int32),
        jax.ShapeDtypeStruct((BATCH, 16), jnp.float32),
    ],
    scratch_shapes=[
        pltpu.VMEM((N_ACC, BATCH, 128), jnp.float32),
        pltpu.VMEM((N_ACC, BATCH, 128), jnp.float32),
    ],
)

_N_WORKERS = 8
_RPW = BATCH // _N_WORKERS


@functools.lru_cache(maxsize=None)
def _make_gather_scale():
    @functools.partial(
        pl.kernel,
        mesh=plsc.VectorSubcoreMesh(core_axis_name="c", subcore_axis_name="s"),
        out_type=jax.ShapeDtypeStruct((BATCH, D_MODEL), jnp.float32),
        scratch_types=[
            pltpu.VMEM((_RPW,), jnp.int32),
            pltpu.VMEM((_RPW, 16), jnp.float32),
            pltpu.VMEM((_RPW, D_MODEL), jnp.float32),
            pltpu.SemaphoreType.DMA,
            pltpu.SemaphoreType.DMA,
        ],
    )
    def _gather_scale(e_hbm, idx_hbm, s_hbm, out_hbm, idx_v, s_v, rows_v, sem,
                      sem2):
        info = plsc.get_sparse_core_info()
        wid = lax.axis_index("s") * info.num_cores + lax.axis_index("c")

        @pl.when(wid < _N_WORKERS)
        def _():
            base = wid * _RPW
            cp_idx = pltpu.async_copy(idx_hbm.at[pl.ds(base, _RPW)], idx_v,
                                      sem)
            cp_s = pltpu.async_copy(s_hbm.at[pl.ds(base, _RPW)], s_v, sem2)
            cp_idx.wait()
            pltpu.async_copy(e_hbm.at[idx_v], rows_v, sem).wait()
            cp_s.wait()
            for i in range(_RPW):
                sbc = s_v[i]
                for jj in range(D_MODEL // 16):
                    sl = pl.ds(jj * 16, 16)
                    rows_v[i, sl] = rows_v[i, sl] * sbc
            pltpu.sync_copy(rows_v, out_hbm.at[pl.ds(base, _RPW)])

    return _gather_scale


def kernel(x, from_logits, E):
    idx, s16 = _argmax_call(x)
    return _make_gather_scale()(E, idx, s16)


# speedup vs baseline: 4.8144x; 1.0203x over previous
import functools

import jax
import jax.numpy as jnp
from jax import lax
from jax.experimental import pallas as pl
from jax.experimental.pallas import tpu as pltpu
from jax.experimental.pallas import tpu_sc as plsc

VOCAB = 100000
D_MODEL = 512
BATCH = 64

V_BLK = 32768
N_BLK = (VOCAB + V_BLK - 1) // V_BLK
CHUNKS = V_BLK // 128
N_ACC = 4
LAST_FULL = (VOCAB - (N_BLK - 1) * V_BLK) // 128
LAST_REM = VOCAB - (N_BLK - 1) * V_BLK - LAST_FULL * 128

_BIG = float(1e9)
_NEG = float("-inf")


def _argmax_body(x_ref, idx_out, s_out, m_ref, c_ref):
    j = pl.program_id(0)

    def scan_chunks(n_full, mask_rem):
        n = n_full + (1 if mask_rem else 0)
        for r in range(BATCH // 8):
            rs = pl.ds(r * 8, 8)
            m = [m_ref[a, rs, :] for a in range(N_ACC)]
            c = [c_ref[a, rs, :] for a in range(N_ACC)]
            for k in range(n):
                a = k % N_ACC
                xc = x_ref[rs, k * 128:(k + 1) * 128]
                if mask_rem and k == n_full:
                    lane = lax.broadcasted_iota(jnp.int32, (8, 128), 1)
                    xc = jnp.where(lane < LAST_REM, xc, _NEG)
                cid = (j * CHUNKS + k).astype(jnp.float32)
                gt = xc > m[a]
                m[a] = jnp.where(gt, xc, m[a])
                c[a] = jnp.where(gt, cid, c[a])
            for a in range(N_ACC):
                m_ref[a, rs, :] = m[a]
                c_ref[a, rs, :] = c[a]

    @pl.when(j == 0)
    def _():
        for a in range(N_ACC):
            m_ref[a] = jnp.full((BATCH, 128), _NEG, jnp.float32)
            c_ref[a] = jnp.zeros((BATCH, 128), jnp.float32)

    @pl.when(j < N_BLK - 1)
    def _():
        scan_chunks(CHUNKS, False)

    @pl.when(j == N_BLK - 1)
    def _():
        scan_chunks(LAST_FULL, True)

        m, c = m_ref[0], c_ref[0]
        for a in range(1, N_ACC):
            ma, ca = m_ref[a], c_ref[a]
            take = (ma > m) | ((ma == m) & (ca < c))
            m = jnp.where(take, ma, m)
            c = jnp.where(take, ca, c)

        rmax = jnp.max(m, axis=1, keepdims=True)
        lane = lax.broadcasted_iota(jnp.int32, (BATCH, 128), 1).astype(
            jnp.float32)
        colf = c * 128.0 + lane
        colmin = jnp.min(jnp.where(m == rmax, colf, _BIG), axis=1,
                         keepdims=True)
        idx_out[...] = jnp.reshape(colmin.astype(jnp.int32), (BATCH,))
        s_out[...] = jnp.broadcast_to((1.0 - rmax) + rmax, (BATCH, 16))


_argmax_call = pl.pallas_call(
    _argmax_body,
    grid=(N_BLK,),
    in_specs=[pl.BlockSpec((BATCH, V_BLK), lambda j: (0, j))],
    out_specs=[
        pl.BlockSpec((BATCH,), lambda j: (0,)),
        pl.BlockSpec((BATCH, 16), lambda j: (0, 0)),
    ],
    out_shape=[
        jax.ShapeDtypeStruct((BATCH,), jnp.int32),
        jax.ShapeDtypeStruct((BATCH, 16), jnp.float32),
    ],
    scratch_shapes=[
        pltpu.VMEM((N_ACC, BATCH, 128), jnp.float32),
        pltpu.VMEM((N_ACC, BATCH, 128), jnp.float32),
    ],
)

_N_WORKERS = 8
_RPW = BATCH // _N_WORKERS


@functools.lru_cache(maxsize=None)
def _make_gather_scale():
    @functools.partial(
        pl.kernel,
        mesh=plsc.VectorSubcoreMesh(core_axis_name="c", subcore_axis_name="s"),
        out_type=jax.ShapeDtypeStruct((BATCH, D_MODEL), jnp.float32),
        scratch_types=[
            pltpu.VMEM((_RPW,), jnp.int32),
            pltpu.VMEM((_RPW, 16), jnp.float32),
            pltpu.VMEM((_RPW, D_MODEL), jnp.float32),
            pltpu.SemaphoreType.DMA,
            pltpu.SemaphoreType.DMA,
        ],
    )
    def _gather_scale(e_hbm, idx_hbm, s_hbm, out_hbm, idx_v, s_v, rows_v, sem,
                      sem2):
        info = plsc.get_sparse_core_info()
        wid = lax.axis_index("s") * info.num_cores + lax.axis_index("c")

        @pl.when(wid < _N_WORKERS)
        def _():
            base = wid * _RPW
            cp_idx = pltpu.async_copy(idx_hbm.at[pl.ds(base, _RPW)], idx_v,
                                      sem)
            cp_s = pltpu.async_copy(s_hbm.at[pl.ds(base, _RPW)], s_v, sem2)
            cp_idx.wait()
            pltpu.async_copy(e_hbm.at[idx_v], rows_v, sem).wait()
            cp_s.wait()
            for i in range(_RPW):
                sbc = s_v[i]
                for jj in range(D_MODEL // 16):
                    sl = pl.ds(jj * 16, 16)
                    rows_v[i, sl] = rows_v[i, sl] * sbc
            pltpu.sync_copy(rows_v, out_hbm.at[pl.ds(base, _RPW)])

    return _gather_scale


def kernel(x, from_logits, E):
    idx, s16 = _argmax_call(x)
    return _make_gather_scale()(E, idx, s16)


# speedup vs baseline: 5.0547x; 1.0499x over previous
import functools

import jax
import jax.numpy as jnp
from jax import lax
from jax.experimental import pallas as pl
from jax.experimental.pallas import tpu as pltpu
from jax.experimental.pallas import tpu_sc as plsc

VOCAB = 100000
D_MODEL = 512
BATCH = 64

V_BLK = 32768
N_BLK = (VOCAB + V_BLK - 1) // V_BLK
CHUNKS = V_BLK // 128
N_ACC = 4
LAST_FULL = (VOCAB - (N_BLK - 1) * V_BLK) // 128
LAST_REM = VOCAB - (N_BLK - 1) * V_BLK - LAST_FULL * 128

_BIG = float(1e9)
_NEG = float("-inf")


def _argmax_body(x_ref, idx_out, s_out, m_ref, c_ref):
    j = pl.program_id(0)

    def scan_chunks(n_full, mask_rem):
        n = n_full + (1 if mask_rem else 0)
        for r in range(BATCH // 8):
            rs = pl.ds(r * 8, 8)
            m = [m_ref[a, rs, :] for a in range(N_ACC)]
            c = [c_ref[a, rs, :] for a in range(N_ACC)]
            for k in range(n):
                a = k % N_ACC
                xc = x_ref[rs, k * 128:(k + 1) * 128]
                if mask_rem and k == n_full:
                    lane = lax.broadcasted_iota(jnp.int32, (8, 128), 1)
                    xc = jnp.where(lane < LAST_REM, xc, _NEG)
                cid = (j * CHUNKS + k).astype(jnp.float32)
                gt = xc > m[a]
                m[a] = jnp.where(gt, xc, m[a])
                c[a] = jnp.where(gt, cid, c[a])
            for a in range(N_ACC):
                m_ref[a, rs, :] = m[a]
                c_ref[a, rs, :] = c[a]

    @pl.when(j == 0)
    def _():
        for a in range(N_ACC):
            m_ref[a] = jnp.full((BATCH, 128), _NEG, jnp.float32)
            c_ref[a] = jnp.zeros((BATCH, 128), jnp.float32)

    @pl.when(j < N_BLK - 1)
    def _():
        scan_chunks(CHUNKS, False)

    @pl.when(j == N_BLK - 1)
    def _():
        scan_chunks(LAST_FULL, True)

        m, c = m_ref[0], c_ref[0]
        for a in range(1, N_ACC):
            ma, ca = m_ref[a], c_ref[a]
            take = (ma > m) | ((ma == m) & (ca < c))
            m = jnp.where(take, ma, m)
            c = jnp.where(take, ca, c)

        rmax = jnp.max(m, axis=1, keepdims=True)
        lane = lax.broadcasted_iota(jnp.int32, (BATCH, 128), 1).astype(
            jnp.float32)
        colf = c * 128.0 + lane
        colmin = jnp.min(jnp.where(m == rmax, colf, _BIG), axis=1,
                         keepdims=True)
        idx_out[...] = jnp.reshape(colmin.astype(jnp.int32), (BATCH,))
        s_out[...] = jnp.broadcast_to((1.0 - rmax) + rmax, (BATCH, 16))


_argmax_call = pl.pallas_call(
    _argmax_body,
    grid=(N_BLK,),
    in_specs=[pl.BlockSpec((BATCH, V_BLK), lambda j: (0, j))],
    out_specs=[
        pl.BlockSpec((BATCH,), lambda j: (0,)),
        pl.BlockSpec((BATCH, 16), lambda j: (0, 0)),
    ],
    out_shape=[
        jax.ShapeDtypeStruct((BATCH,), jnp.int32),
        jax.ShapeDtypeStruct((BATCH, 16), jnp.float32),
    ],
    scratch_shapes=[
        pltpu.VMEM((N_ACC, BATCH, 128), jnp.float32),
        pltpu.VMEM((N_ACC, BATCH, 128), jnp.float32),
    ],
)

_N_WORKERS = 8
_RPW = BATCH // _N_WORKERS


@functools.lru_cache(maxsize=None)
def _make_gather_scale():
    @functools.partial(
        pl.kernel,
        mesh=plsc.VectorSubcoreMesh(core_axis_name="c", subcore_axis_name="s", num_cores=1),
        out_type=jax.ShapeDtypeStruct((BATCH, D_MODEL), jnp.float32),
        scratch_types=[
            pltpu.VMEM((_RPW,), jnp.int32),
            pltpu.VMEM((_RPW, 16), jnp.float32),
            pltpu.VMEM((_RPW, D_MODEL), jnp.float32),
            pltpu.SemaphoreType.DMA,
            pltpu.SemaphoreType.DMA,
        ],
    )
    def _gather_scale(e_hbm, idx_hbm, s_hbm, out_hbm, idx_v, s_v, rows_v, sem,
                      sem2):
        wid = lax.axis_index("s")

        @pl.when(wid < _N_WORKERS)
        def _():
            base = wid * _RPW
            cp_idx = pltpu.async_copy(idx_hbm.at[pl.ds(base, _RPW)], idx_v,
                                      sem)
            cp_s = pltpu.async_copy(s_hbm.at[pl.ds(base, _RPW)], s_v, sem2)
            cp_idx.wait()
            pltpu.async_copy(e_hbm.at[idx_v], rows_v, sem).wait()
            cp_s.wait()
            for i in range(_RPW):
                sbc = s_v[i]
                for jj in range(D_MODEL // 16):
                    sl = pl.ds(jj * 16, 16)
                    rows_v[i, sl] = rows_v[i, sl] * sbc
            pltpu.sync_copy(rows_v, out_hbm.at[pl.ds(base, _RPW)])

    return _gather_scale


def kernel(x, from_logits, E):
    idx, s16 = _argmax_call(x)
    return _make_gather_scale()(E, idx, s16)
